# baseline (speedup 1.0000x reference)
import functools

import jax
import jax.numpy as jnp
from jax.experimental import pallas as pl
from jax.experimental.pallas import tpu as pltpu


def _dcn_block(e_ref, c_ref, lab_ref, loss_ref, part_ref, *, n_total):
    e = e_ref[...]
    c = c_ref[...]
    k, bn = part_ref.shape

    xct = jax.lax.dot_general(
        (c * -2.0).astype(jnp.bfloat16),
        e.astype(jnp.bfloat16),
        (((1,), (1,)), ((), ())),
        preferred_element_type=jnp.float32,
    )
    c2 = jnp.sum(c * c, axis=1, keepdims=True)
    part_ref[...] = xct + c2

    si = jax.lax.broadcasted_iota(jnp.int32, (8, bn), 0)

    def body(r, carry):
        bv, bi = carry
        v = part_ref[pl.ds(r * 8, 8), :]
        m = v < bv
        bv = jnp.where(m, v, bv)
        bi = jnp.where(m, si + r * 8, bi)
        return bv, bi

    bv, bi = jax.lax.fori_loop(1, k // 8, body, (part_ref[0:8, :], si))

    minv = jnp.min(bv, axis=0, keepdims=True)
    lab = jnp.min(jnp.where(bv == minv, bi, k), axis=0, keepdims=True)
    lab_ref[0] = lab

    blk = jnp.sum(e * e) + jnp.sum(minv)
    loss_ref[...] = (blk * (1.0 / n_total)).reshape(1, 1, 1)


def kernel(embedded, centers):
    n, d = embedded.shape
    k = centers.shape[0]
    bn = 1024
    g = n // bn

    lab3, loss_parts = pl.pallas_call(
        functools.partial(_dcn_block, n_total=n),
        grid=(g,),
        in_specs=[
            pl.BlockSpec((bn, d), lambda i: (i, 0)),
            pl.BlockSpec((k, d), lambda i: (0, 0)),
        ],
        out_specs=[
            pl.BlockSpec((1, 1, bn), lambda i: (i, 0, 0)),
            pl.BlockSpec((1, 1, 1), lambda i: (i, 0, 0)),
        ],
        out_shape=[
            jax.ShapeDtypeStruct((g, 1, bn), jnp.int32),
            jax.ShapeDtypeStruct((g, 1, 1), jnp.float32),
        ],
        scratch_shapes=[pltpu.VMEM((k, bn), jnp.float32)],
        compiler_params=pltpu.CompilerParams(
            dimension_semantics=("parallel",),
        ),
    )(embedded, centers)

    return lab3.reshape(n), jnp.sum(loss_parts)


# speedup vs baseline: 1.6613x; 1.6613x over previous
import functools

import jax
import jax.numpy as jnp
from jax.experimental import pallas as pl
from jax.experimental.pallas import tpu as pltpu


def _dcn_block(e_ref, c_ref, lab_ref, loss_ref, *, n_total):
    e = e_ref[...]
    c = c_ref[...]
    k = c.shape[0]

    xct = jax.lax.dot_general(
        (c * -2.0).astype(jnp.bfloat16),
        e.astype(jnp.bfloat16),
        (((1,), (1,)), ((), ())),
        preferred_element_type=jnp.float32,
    )
    c2 = jnp.sum(c * c, axis=1, keepdims=True)
    part = xct + c2

    minv = jnp.min(part, axis=0, keepdims=True)
    iota = jax.lax.broadcasted_iota(jnp.int32, part.shape, 0).astype(jnp.float32)
    labf = jnp.min(jnp.where(part == minv, iota, float(k)), axis=0, keepdims=True)
    lab_ref[0] = labf.astype(jnp.int32)

    blk = jnp.sum(e * e) + jnp.sum(minv)
    loss_ref[...] = (blk * (1.0 / n_total)).reshape(1, 1, 1)


def kernel(embedded, centers):
    n, d = embedded.shape
    k = centers.shape[0]
    bn = 1024
    g = n // bn

    lab3, loss_parts = pl.pallas_call(
        functools.partial(_dcn_block, n_total=n),
        grid=(g,),
        in_specs=[
            pl.BlockSpec((bn, d), lambda i: (i, 0)),
            pl.BlockSpec((k, d), lambda i: (0, 0)),
        ],
        out_specs=[
            pl.BlockSpec((1, 1, bn), lambda i: (i, 0, 0)),
            pl.BlockSpec((1, 1, 1), lambda i: (i, 0, 0)),
        ],
        out_shape=[
            jax.ShapeDtypeStruct((g, 1, bn), jnp.int32),
            jax.ShapeDtypeStruct((g, 1, 1), jnp.float32),
        ],
        compiler_params=pltpu.CompilerParams(
            dimension_semantics=("parallel",),
        ),
    )(embedded, centers)

    return lab3.reshape(n), jnp.sum(loss_parts)


# speedup vs baseline: 1.9035x; 1.1458x over previous
import functools

import jax
import jax.numpy as jnp
from jax.experimental import pallas as pl
from jax.experimental.pallas import tpu as pltpu


def _dcn_block(e_ref, c_ref, lab_ref, loss_ref, *, n_total):
    e = e_ref[...]
    c = c_ref[...]
    k = c.shape[0]

    xct = jax.lax.dot_general(
        (c * -2.0).astype(jnp.bfloat16),
        e.astype(jnp.bfloat16),
        (((1,), (1,)), ((), ())),
        preferred_element_type=jnp.float32,
    )
    c2 = jnp.sum(c * c, axis=1, keepdims=True)
    part = xct + c2

    minv = jnp.min(part, axis=0, keepdims=True)
    iota = jax.lax.broadcasted_iota(jnp.int32, part.shape, 0).astype(jnp.float32)
    labf = jnp.min(jnp.where(part == minv, iota, float(k)), axis=0, keepdims=True)
    lab_ref[0] = labf.astype(jnp.int32)

    blk = jnp.sum(e * e) + jnp.sum(minv)
    loss_ref[...] = (blk * (1.0 / n_total)).reshape(1, 1, 1)


def kernel(embedded, centers):
    n, d = embedded.shape
    k = centers.shape[0]
    bn = 4096
    g = n // bn

    lab3, loss_parts = pl.pallas_call(
        functools.partial(_dcn_block, n_total=n),
        grid=(g,),
        in_specs=[
            pl.BlockSpec((bn, d), lambda i: (i, 0)),
            pl.BlockSpec((k, d), lambda i: (0, 0)),
        ],
        out_specs=[
            pl.BlockSpec((1, 1, bn), lambda i: (i, 0, 0)),
            pl.BlockSpec((1, 1, 1), lambda i: (i, 0, 0)),
        ],
        out_shape=[
            jax.ShapeDtypeStruct((g, 1, bn), jnp.int32),
            jax.ShapeDtypeStruct((g, 1, 1), jnp.float32),
        ],
        compiler_params=pltpu.CompilerParams(
            dimension_semantics=("parallel",),
        ),
    )(embedded, centers)

    return lab3.reshape(n), jnp.sum(loss_parts)
